# baseline (speedup 1.0000x reference)
import functools

import jax
import jax.numpy as jnp
from jax import lax
from jax.experimental import pallas as pl
from jax.experimental.pallas import tpu as pltpu
from jax.experimental.pallas import tpu_sc as plsc

N = 256 * 256
K = 8
C = 128
CW = C // 2
OUT_SIZES = (256, 256, 128)

NC = 2
NS = 16
LANES = 16
NW = NC * NS
RPW = N // NW
BROWS = 64
NCHUNK = RPW // BROWS
PAIRS = BROWS * K
IDXROWS = PAIRS // 128
PB = 64


def _knn_sc(Gp, idxm, wm):
    mesh = plsc.VectorSubcoreMesh(core_axis_name="c", subcore_axis_name="s")

    @functools.partial(
        pl.kernel,
        mesh=mesh,
        out_type=(jax.ShapeDtypeStruct(OUT_SIZES, jnp.float32),
                  jax.ShapeDtypeStruct((NC, N, CW), jnp.int32)),
        compiler_params=pltpu.CompilerParams(use_tc_tiling_on_sc=False),
        scratch_types=[
            pltpu.VMEM((2, IDXROWS, 128), jnp.int32),
            pltpu.VMEM((IDXROWS, 128), jnp.float32),
            pltpu.VMEM((IDXROWS, 128), jnp.float32),
            pltpu.VMEM((2, PAIRS, CW), jnp.int32),
            pltpu.VMEM((2, BROWS, C), jnp.float32),
            pltpu.VMEM((2, PB, C), jnp.float32),
            pltpu.VMEM((2, PB, CW), jnp.int32),
            pltpu.SemaphoreType.DMA,
            pltpu.SemaphoreType.DMA,
            pltpu.SemaphoreType.DMA,
            pltpu.SemaphoreType.DMA,
            pltpu.SemaphoreType.DMA,
            pltpu.SemaphoreType.DMA,
            pltpu.SemaphoreType.DMA,
            pltpu.SemaphoreType.DMA,
        ],
    )
    def k(Gf_hbm, idx_hbm, w_hbm, out_hbm, Gs_hbm, idx_v, w_v0, w_v1, rows_v,
          out_v, pin_v, pout_v, gsem0, gsem1, osem0, osem1, pisem0, pisem1,
          posem0, posem1):
        cidx = lax.axis_index("c")
        sidx = lax.axis_index("s")
        wid = lax.axis_index("s") * NC + lax.axis_index("c")

        G_hbm = Gs_hbm.at[cidx]
        prow0 = sidx * (N // NS)
        pisems = (pisem0, pisem1)
        posems = (posem0, posem1)
        NPCH = N // NS // PB

        def pack_in(i, b):
            pltpu.async_copy(Gf_hbm.at[pl.ds(prow0 + i * PB, PB)],
                             pin_v.at[b], pisems[b])

        def pack_chunk(i, b):
            pltpu.make_async_copy(Gf_hbm.at[pl.ds(prow0 + i * PB, PB)],
                                  pin_v.at[b], pisems[b]).wait()

            @pl.when(i >= 2)
            def _drain_pack_out():
                pltpu.make_async_copy(
                    pout_v.at[b],
                    G_hbm.at[pl.ds(prow0 + (i - 2) * PB, PB)],
                    posems[b]).wait()

            def prow(r, rc):
                for j in range(CW // LANES):
                    va = lax.bitcast_convert_type(
                        pin_v[b, r, pl.ds(j * LANES, LANES)], jnp.int32)
                    vb = lax.bitcast_convert_type(
                        pin_v[b, r, pl.ds(CW + j * LANES, LANES)], jnp.int32)
                    half = jnp.int32(32768)
                    wa = lax.shift_right_logical(va + half, 16)
                    wb = lax.bitwise_and(vb + half, jnp.int32(-65536))
                    pout_v[b, r, pl.ds(j * LANES, LANES)] = (
                        jnp.bitwise_or(wa, wb))
                return rc

            lax.fori_loop(0, PB, prow, 0)
            pltpu.async_copy(pout_v.at[b],
                             G_hbm.at[pl.ds(prow0 + i * PB, PB)], posems[b])

        pack_in(0, 0)
        pack_in(1, 1)

        def pack_body(t, carry):
            for b in range(2):
                i = 2 * t + b
                pack_chunk(i, b)

                @pl.when(i + 2 < NPCH)
                def _prefetch_pack():
                    pack_in(i + 2, b)
            return carry

        lax.fori_loop(0, NPCH // 2, pack_body, 0)
        for b in range(2):
            pltpu.make_async_copy(
                pout_v.at[b],
                G_hbm.at[pl.ds(prow0 + (NPCH - 2 + b) * PB, PB)],
                posems[b]).wait()
        plsc.subcore_barrier()

        row0 = wid * RPW
        irow0 = wid * (RPW * K // 128)

        def out_slice(g):
            r = row0 + g * BROWS
            return out_hbm.at[lax.shift_right_logical(r, 8),
                              pl.ds(lax.bitwise_and(r, 255), BROWS)]
        gsems = (gsem0, gsem1)
        osems = (osem0, osem1)
        wvs_ref = (w_v0, w_v1)

        def issue_fetch(g, b):
            ir = irow0 + g * IDXROWS
            pltpu.sync_copy(idx_hbm.at[pl.ds(ir, IDXROWS)], idx_v.at[b])
            for h in range(IDXROWS):
                pltpu.async_copy(G_hbm.at[idx_v.at[b, h]],
                                 rows_v.at[b, pl.ds(h * 128, 128)], gsems[b])
            pltpu.async_copy(w_hbm.at[pl.ds(ir, IDXROWS)], wvs_ref[b],
                             gsems[b])

        def wait_fetch(g, b):
            ir = irow0 + g * IDXROWS
            for h in range(IDXROWS):
                pltpu.make_async_copy(G_hbm.at[idx_v.at[b, h]],
                                      rows_v.at[b, pl.ds(h * 128, 128)],
                                      gsems[b]).wait()
            pltpu.make_async_copy(w_hbm.at[pl.ds(ir, IDXROWS)], wvs_ref[b],
                                  gsems[b]).wait()

        issue_fetch(0, 0)

        def chunk_body(t, carry):
            for b in range(2):
                g = 2 * t + b
                if b == 0:
                    issue_fetch(g + 1, 1)
                else:
                    @pl.when(t < NCHUNK // 2 - 1)
                    def _prefetch():
                        issue_fetch(g + 1, 0)
                wait_fetch(g, b)

                @pl.when(t >= 1)
                def _drain_store():
                    pltpu.make_async_copy(out_v.at[b], out_slice(g - 2),
                                          osems[b]).wait()

                def pair_body(rr, rc):
                    f16 = rr * 2 * K
                    wrow = wvs_ref[b][lax.shift_right_logical(f16, 7),
                                      pl.ds(lax.bitwise_and(f16, 127), LANES)]
                    for s in range(2):
                        r = rr * 2 + s
                        p0 = f16 + s * K
                        accA = [None] * (CW // LANES)
                        accB = [None] * (CW // LANES)
                        for kk in range(K):
                            wv = jnp.full((LANES,), wrow[s * K + kk],
                                          jnp.float32) * (1.0 / K)
                            for j in range(CW // LANES):
                                v = rows_v[b, p0 + kk,
                                           pl.ds(j * LANES, LANES)]
                                va = lax.bitcast_convert_type(
                                    lax.shift_left(v, 16), jnp.float32)
                                vb = lax.bitcast_convert_type(
                                    lax.bitwise_and(v, jnp.int32(-65536)),
                                    jnp.float32)
                                if kk == 0:
                                    accA[j] = va * wv
                                    accB[j] = vb * wv
                                else:
                                    accA[j] = accA[j] + va * wv
                                    accB[j] = accB[j] + vb * wv
                        for j in range(CW // LANES):
                            out_v[b, r, pl.ds(j * LANES, LANES)] = accA[j]
                            out_v[b, r, pl.ds(CW + j * LANES, LANES)] = accB[j]
                    return rc

                lax.fori_loop(0, BROWS // 2, pair_body, 0)
                pltpu.async_copy(out_v.at[b], out_slice(g), osems[b])
            return carry

        lax.fori_loop(0, NCHUNK // 2, chunk_body, 0)

        for b in range(2):
            pltpu.make_async_copy(out_v.at[b], out_slice(NCHUNK - 2 + b),
                                  osems[b]).wait()

    return k(Gp, idxm, wm)[0]


def kernel(x, G, neighbor_index, neighbor_dist):
    del x
    idxm = neighbor_index.astype(jnp.int32).reshape(N * K // 128, 128)
    wm = neighbor_dist.astype(jnp.float32).reshape(N * K // 128, 128)
    return _knn_sc(G, idxm, wm)


# speedup vs baseline: 1.4355x; 1.4355x over previous
import functools

import jax
import jax.numpy as jnp
from jax import lax
from jax.experimental import pallas as pl
from jax.experimental.pallas import tpu as pltpu
from jax.experimental.pallas import tpu_sc as plsc

N = 256 * 256
K = 8
C = 128
CW = C // 2
OUT_SIZES = (256, 256, 128)

NC = 2
NS = 16
LANES = 16
NW = NC * NS
RPW = N // NW
BROWS = 64
NCHUNK = RPW // BROWS
PAIRS = BROWS * K
IDXROWS = PAIRS // 128


PB = 64
PROWS = N // NW
NPCH = PROWS // PB


def _pack_sc(G):
    mesh = plsc.VectorSubcoreMesh(core_axis_name="c", subcore_axis_name="s")

    @functools.partial(
        pl.kernel,
        mesh=mesh,
        out_type=jax.ShapeDtypeStruct((N, CW), jnp.int32),
        compiler_params=pltpu.CompilerParams(use_tc_tiling_on_sc=False),
        scratch_types=[
            pltpu.VMEM((2, PB, C), jnp.float32),
            pltpu.VMEM((2, PB, CW), jnp.int32),
            pltpu.SemaphoreType.DMA,
            pltpu.SemaphoreType.DMA,
            pltpu.SemaphoreType.DMA,
            pltpu.SemaphoreType.DMA,
        ],
    )
    def k(Gf_hbm, Gp_hbm, pin_v, pout_v, pisem0, pisem1, posem0, posem1):
        wid = lax.axis_index("s") * NC + lax.axis_index("c")
        prow0 = wid * PROWS
        pisems = (pisem0, pisem1)
        posems = (posem0, posem1)

        def pack_in(i, b):
            pltpu.async_copy(Gf_hbm.at[pl.ds(prow0 + i * PB, PB)],
                             pin_v.at[b], pisems[b])

        def pack_chunk(i, b):
            pltpu.make_async_copy(Gf_hbm.at[pl.ds(prow0 + i * PB, PB)],
                                  pin_v.at[b], pisems[b]).wait()

            @pl.when(i >= 2)
            def _drain_out():
                pltpu.make_async_copy(
                    pout_v.at[b],
                    Gp_hbm.at[pl.ds(prow0 + (i - 2) * PB, PB)],
                    posems[b]).wait()

            def prow(r, rc):
                half = jnp.int32(32768)
                for j in range(CW // LANES):
                    va = lax.bitcast_convert_type(
                        pin_v[b, r, pl.ds(j * LANES, LANES)], jnp.int32)
                    vb = lax.bitcast_convert_type(
                        pin_v[b, r, pl.ds(CW + j * LANES, LANES)], jnp.int32)
                    wa = lax.shift_right_logical(va + half, 16)
                    wb = lax.bitwise_and(vb + half, jnp.int32(-65536))
                    pout_v[b, r, pl.ds(j * LANES, LANES)] = (
                        jnp.bitwise_or(wa, wb))
                return rc

            lax.fori_loop(0, PB, prow, 0)
            pltpu.async_copy(pout_v.at[b],
                             Gp_hbm.at[pl.ds(prow0 + i * PB, PB)], posems[b])

        pack_in(0, 0)
        pack_in(1, 1)

        def pack_body(t, carry):
            for b in range(2):
                i = 2 * t + b
                pack_chunk(i, b)

                @pl.when(i + 2 < NPCH)
                def _prefetch():
                    pack_in(i + 2, b)
            return carry

        lax.fori_loop(0, NPCH // 2, pack_body, 0)
        for b in range(2):
            pltpu.make_async_copy(
                pout_v.at[b],
                Gp_hbm.at[pl.ds(prow0 + (NPCH - 2 + b) * PB, PB)],
                posems[b]).wait()

    return k(G)


def _knn_sc(Gp, idxm, wm):
    mesh = plsc.VectorSubcoreMesh(core_axis_name="c", subcore_axis_name="s")

    @functools.partial(
        pl.kernel,
        mesh=mesh,
        out_type=jax.ShapeDtypeStruct(OUT_SIZES, jnp.float32),
        compiler_params=pltpu.CompilerParams(use_tc_tiling_on_sc=False),
        scratch_types=[
            pltpu.VMEM((2, IDXROWS, 128), jnp.int32),
            pltpu.VMEM((IDXROWS, 128), jnp.float32),
            pltpu.VMEM((IDXROWS, 128), jnp.float32),
            pltpu.VMEM((2, PAIRS, CW), jnp.int32),
            pltpu.VMEM((2, BROWS, C), jnp.float32),
            pltpu.SemaphoreType.DMA,
            pltpu.SemaphoreType.DMA,
            pltpu.SemaphoreType.DMA,
            pltpu.SemaphoreType.DMA,
        ],
    )
    def k(G_hbm, idx_hbm, w_hbm, out_hbm, idx_v, w_v0, w_v1, rows_v, out_v,
          gsem0, gsem1, osem0, osem1):
        wid = lax.axis_index("s") * NC + lax.axis_index("c")
        row0 = wid * RPW
        irow0 = wid * (RPW * K // 128)

        def out_slice(g):
            r = row0 + g * BROWS
            return out_hbm.at[lax.shift_right_logical(r, 8),
                              pl.ds(lax.bitwise_and(r, 255), BROWS)]
        gsems = (gsem0, gsem1)
        osems = (osem0, osem1)
        wvs_ref = (w_v0, w_v1)

        def issue_fetch(g, b):
            ir = irow0 + g * IDXROWS
            pltpu.sync_copy(idx_hbm.at[pl.ds(ir, IDXROWS)], idx_v.at[b])
            for h in range(IDXROWS):
                pltpu.async_copy(G_hbm.at[idx_v.at[b, h]],
                                 rows_v.at[b, pl.ds(h * 128, 128)], gsems[b])
            pltpu.async_copy(w_hbm.at[pl.ds(ir, IDXROWS)], wvs_ref[b],
                             gsems[b])

        def wait_fetch(g, b):
            ir = irow0 + g * IDXROWS
            for h in range(IDXROWS):
                pltpu.make_async_copy(G_hbm.at[idx_v.at[b, h]],
                                      rows_v.at[b, pl.ds(h * 128, 128)],
                                      gsems[b]).wait()
            pltpu.make_async_copy(w_hbm.at[pl.ds(ir, IDXROWS)], wvs_ref[b],
                                  gsems[b]).wait()

        issue_fetch(0, 0)

        def chunk_body(t, carry):
            for b in range(2):
                g = 2 * t + b
                if b == 0:
                    issue_fetch(g + 1, 1)
                else:
                    @pl.when(t < NCHUNK // 2 - 1)
                    def _prefetch():
                        issue_fetch(g + 1, 0)
                wait_fetch(g, b)

                @pl.when(t >= 1)
                def _drain_store():
                    pltpu.make_async_copy(out_v.at[b], out_slice(g - 2),
                                          osems[b]).wait()

                def pair_body(rr, rc):
                    f16 = rr * 2 * K
                    wrow = wvs_ref[b][lax.shift_right_logical(f16, 7),
                                      pl.ds(lax.bitwise_and(f16, 127), LANES)]
                    for s in range(2):
                        r = rr * 2 + s
                        p0 = f16 + s * K
                        accA = [None] * (CW // LANES)
                        accB = [None] * (CW // LANES)
                        for kk in range(K):
                            wv = jnp.full((LANES,), wrow[s * K + kk],
                                          jnp.float32) * (1.0 / K)
                            for j in range(CW // LANES):
                                v = rows_v[b, p0 + kk,
                                           pl.ds(j * LANES, LANES)]
                                va = lax.bitcast_convert_type(
                                    lax.shift_left(v, 16), jnp.float32)
                                vb = lax.bitcast_convert_type(
                                    lax.bitwise_and(v, jnp.int32(-65536)),
                                    jnp.float32)
                                if kk == 0:
                                    accA[j] = va * wv
                                    accB[j] = vb * wv
                                else:
                                    accA[j] = accA[j] + va * wv
                                    accB[j] = accB[j] + vb * wv
                        for j in range(CW // LANES):
                            out_v[b, r, pl.ds(j * LANES, LANES)] = accA[j]
                            out_v[b, r, pl.ds(CW + j * LANES, LANES)] = accB[j]
                    return rc

                lax.fori_loop(0, BROWS // 2, pair_body, 0)
                pltpu.async_copy(out_v.at[b], out_slice(g), osems[b])
            return carry

        lax.fori_loop(0, NCHUNK // 2, chunk_body, 0)

        for b in range(2):
            pltpu.make_async_copy(out_v.at[b], out_slice(NCHUNK - 2 + b),
                                  osems[b]).wait()

    return k(Gp, idxm, wm)


def kernel(x, G, neighbor_index, neighbor_dist):
    del x
    idxm = neighbor_index.astype(jnp.int32).reshape(N * K // 128, 128)
    wm = neighbor_dist.astype(jnp.float32).reshape(N * K // 128, 128)
    return _knn_sc(_pack_sc(G), idxm, wm)
